# baseline (speedup 1.0000x reference)
import functools

import jax
import jax.numpy as jnp
import numpy as np
from jax import lax
from jax.experimental import pallas as pl
from jax.experimental.pallas import tpu as pltpu
from jax.experimental.pallas import tpu_sc as plsc

B = 4096
D = 128
S = 50
SP = 64
NW = 32
BPW = B // NW
CPW = BPW // 2
CHUNK = 2 * SP
WIDS = BPW * SP


def _mask_at(idx_ref, pos):
    v = idx_ref[pl.ds(pos, 16)]
    return jnp.where(v[0] != 0, jnp.float32(1.0), jnp.float32(0.0))


NBUF = 2
INV50 = 1.0 / 50.0


def _sc_body(bk1, bk2, bk_t,
             co_i, co_t, st_i, st_t, zi_i, zi_t, te_i, te_t, sc_i, sc_t,
             bk_o, co_o, st_o, zi_o, te_o, sc_o, n0_o,
             idx1d, idx2d, rows0, rows1, rows2, rows3, acc, acc2, n0buf,
             sidx0, sidx1, sidx2, sidx3, sidx4,
             sem0, sem1, sem2, sem3):
    wid = lax.axis_index("s") * 2 + lax.axis_index("c")
    base = wid * BPW
    rows = (rows0, rows1, rows2, rows3)
    sems = (sem0, sem1, sem2, sem3)
    lanes = lax.iota(jnp.int32, 16)

    def do_seq(flat_hbm, ids2d_hbm, table_hbm, out_hbm, n0_hbm):
        pltpu.sync_copy(flat_hbm.at[pl.ds(base * SP, WIDS)],
                        idx1d.at[pl.ds(0, WIDS)])
        pltpu.sync_copy(ids2d_hbm.at[pl.ds(wid * CPW, CPW)], idx2d)

        for b in range(NBUF):
            pltpu.make_async_copy(
                table_hbm.at[idx2d.at[b]], rows[b], sems[b]).start()

        def group(g, _):
            for b in range(NBUF):
                c = g * NBUF + b
                rb = rows[b]
                pltpu.make_async_copy(
                    table_hbm.at[idx2d.at[0]], rb, sems[b]).wait()
                for half in range(2):
                    r = c * 2 + half
                    base_i = c * CHUNK + half * SP

                    def jbody(j, accs, _rb=rb, _half=half):
                        gg = _half * SP + j
                        return tuple(
                            accs[k] + _rb[gg, pl.ds(k * 16, 16)]
                            for k in range(8))

                    accs = lax.fori_loop(
                        0, SP, jbody,
                        tuple(jnp.zeros((16,), jnp.float32) for _ in range(8)),
                        unroll=4)
                    n0 = jnp.zeros((16,), jnp.int32)
                    for t in range(3):
                        v = idx1d[pl.ds(base_i + t * 16, 16)]
                        n0 = n0 + plsc.all_reduce_population_count(v == 0)
                    v3 = idx1d[pl.ds(base_i + 48, 16)]
                    n0 = n0 + plsc.all_reduce_population_count(
                        (v3 == 0) & (lanes < SP - 48))
                    n0buf[r, pl.ds(0, 16)] = n0.astype(jnp.float32)
                    for k in range(8):
                        acc[r, pl.ds(k * 16, 16)] = (
                            accs[k] * jnp.float32(INV50))
                nxt = c + NBUF

                @pl.when(nxt < CPW)
                def _():
                    pltpu.make_async_copy(
                        table_hbm.at[idx2d.at[nxt]], rb, sems[b]).start()
            return 0

        lax.fori_loop(0, CPW // NBUF, group, 0)
        pltpu.sync_copy(acc, out_hbm.at[pl.ds(base, BPW)])
        pltpu.sync_copy(n0buf, n0_hbm.at[pl.ds(base, BPW)])

    def mask_rows(buf, idx_ref):
        def rbody(g, _):
            m = _mask_at(idx_ref, g)
            for k in range(8):
                buf[g, pl.ds(k * 16, 16)] = buf[g, pl.ds(k * 16, 16)] * m
            return 0
        lax.fori_loop(0, BPW, rbody, 0)

    do_seq(bk1, bk2, bk_t, bk_o, n0_o)

    singles = ((co_i, co_t, co_o, sidx0), (st_i, st_t, st_o, sidx1),
               (zi_i, zi_t, zi_o, sidx2), (te_i, te_t, te_o, sidx3),
               (sc_i, sc_t, sc_o, sidx4))
    bufs = (acc, acc2)
    for f, (ids_hbm, _, _, sx) in enumerate(singles):
        pltpu.sync_copy(ids_hbm.at[pl.ds(base, BPW)], sx.at[pl.ds(0, BPW)])
    pltpu.make_async_copy(
        singles[0][1].at[sidx0.at[pl.ds(0, BPW)]], bufs[0], sems[0]).start()
    for f, (_, table_hbm, out_hbm, sx) in enumerate(singles):
        buf = bufs[f % 2]
        pltpu.make_async_copy(
            table_hbm.at[sx.at[pl.ds(0, BPW)]], buf, sems[f % 2]).wait()
        if f + 1 < len(singles):
            nx = singles[f + 1]
            pltpu.make_async_copy(
                nx[1].at[nx[3].at[pl.ds(0, BPW)]],
                bufs[(f + 1) % 2], sems[(f + 1) % 2]).start()
        mask_rows(buf, sx)
        pltpu.sync_copy(buf, out_hbm.at[pl.ds(base, BPW)])


@functools.cache
def _sc_pool():
    return pl.kernel(
        _sc_body,
        out_type=[jax.ShapeDtypeStruct((B, D), jnp.float32) for _ in range(6)]
        + [jax.ShapeDtypeStruct((B, 16), jnp.float32)],
        mesh=plsc.VectorSubcoreMesh(core_axis_name="c", subcore_axis_name="s"),
        scratch_types=(
            [pltpu.VMEM((WIDS + 16,), jnp.int32),
             pltpu.VMEM((CPW, CHUNK), jnp.int32)]
            + [pltpu.VMEM((CHUNK, D) if b < NBUF else (8, D), jnp.float32)
               for b in range(4)]
            + [pltpu.VMEM((BPW, D), jnp.float32),
               pltpu.VMEM((BPW, D), jnp.float32),
               pltpu.VMEM((BPW, 16), jnp.float32)]
            + [pltpu.VMEM((BPW + 16,), jnp.int32) for _ in range(5)]
            + [pltpu.SemaphoreType.DMA for _ in range(4)]
        ),
        compiler_params=pltpu.CompilerParams(needs_layout_passes=False),
    )


V_SMALL = 1000
CBLK = 128


def _counts_pool_body(th_ids, ca_ids, sk_ids, th_t, ca_t, sk_t,
                      th_o, ca_o, sk_o, cnt):
    vcol = lax.broadcasted_iota(jnp.int32, (CBLK, V_SMALL), 1)
    colmask = (vcol > 0).astype(jnp.float32)
    for ids_ref, t_ref, o_ref in ((th_ids, th_t, th_o),
                                  (ca_ids, ca_t, ca_o),
                                  (sk_ids, sk_t, sk_o)):
        cnt[...] = jnp.zeros((CBLK, V_SMALL), jnp.float32)
        for j in range(S):
            cnt[...] = cnt[...] + (
                ids_ref[:, j:j + 1] == vcol).astype(jnp.float32)
        o_ref[...] = jnp.dot(cnt[...] * colmask, t_ref[...],
                             preferred_element_type=jnp.float32) * (1.0 / S)


def _counts_pool(th_ids, ca_ids, sk_ids, th_t, ca_t, sk_t):
    grid = (B // CBLK,)
    return pl.pallas_call(
        _counts_pool_body,
        grid=grid,
        in_specs=[pl.BlockSpec((CBLK, S), lambda i: (i, 0))] * 3
        + [pl.BlockSpec((V_SMALL, D), lambda i: (0, 0))] * 3,
        out_specs=[pl.BlockSpec((CBLK, D), lambda i: (i, 0))] * 3,
        out_shape=[jax.ShapeDtypeStruct((B, D), jnp.float32)] * 3,
        scratch_shapes=[pltpu.VMEM((CBLK, V_SMALL), jnp.float32)],
    )(th_ids, ca_ids, sk_ids, th_t, ca_t, sk_t)


def _mlp_body(p_refs, uf_ref, n0_ref, t0p_ref, w1_ref, b1_ref, w2_ref,
              b2_ref, out_ref):
    h = jnp.dot(uf_ref[...], w1_ref[pl.ds(9 * D, D), :],
                preferred_element_type=jnp.float32)
    for f in range(9):
        h = h + jnp.dot(p_refs[f][...], w1_ref[pl.ds(f * D, D), :],
                        preferred_element_type=jnp.float32)
    t0w = jnp.dot(t0p_ref[0:1, :], w1_ref[pl.ds(0, D), :],
                  preferred_element_type=jnp.float32)
    h = h - (n0_ref[:, 0:1] * jnp.float32(INV50)) * t0w
    h = jnp.maximum(h + b1_ref[0:1, :], 0.0)
    out_ref[...] = (jnp.dot(h, w2_ref[...], preferred_element_type=jnp.float32)
                    + b2_ref[0:1, :])


def _mlp(pooled, uf, n0, t0p, w1, b1, w2, b2):
    blk = 512
    grid = (B // blk,)
    p_specs = [pl.BlockSpec((blk, D), lambda i: (i, 0)) for _ in range(9)]

    def body(*refs):
        _mlp_body(list(refs[0:9]), refs[9], refs[10], refs[11], refs[12],
                  refs[13], refs[14], refs[15], refs[16])

    return pl.pallas_call(
        body,
        grid=grid,
        in_specs=p_specs + [
            pl.BlockSpec((blk, D), lambda i: (i, 0)),
            pl.BlockSpec((blk, 16), lambda i: (i, 0)),
            pl.BlockSpec((8, D), lambda i: (0, 0)),
            pl.BlockSpec((10 * D, 256), lambda i: (0, 0)),
            pl.BlockSpec((8, 256), lambda i: (0, 0)),
            pl.BlockSpec((256, 64), lambda i: (0, 0)),
            pl.BlockSpec((8, 64), lambda i: (0, 0)),
        ],
        out_specs=pl.BlockSpec((blk, 64), lambda i: (i, 0)),
        out_shape=jax.ShapeDtypeStruct((B, 64), jnp.float32),
    )(*pooled, uf, n0, t0p, w1, b1, w2, b2)


def kernel(last_book_ids, last_book_mask, last_theme_ids, last_theme_mask,
           last_category_ids, last_category_mask, last_reading_skills_id,
           last_reading_skills_mask, country_ids, country_mask, state_ids,
           state_mask, zipcode_ids, zipcode_mask, teacher_ids, teacher_mask,
           school_ids, school_mask, user_features, book_table, theme_table,
           category_table, skill_table, country_table, state_table,
           zipcode_table, teacher_table, school_table, W1, b1, W2, b2):
    def seq_prep(ids):
        p = jnp.pad(ids.astype(jnp.int32), ((0, 0), (0, SP - S)))
        return p.reshape(-1), p.reshape(B // 2, CHUNK)

    bk1, bk2 = seq_prep(last_book_ids)
    co = country_ids.astype(jnp.int32).reshape(-1)
    st = state_ids.astype(jnp.int32).reshape(-1)
    zi = zipcode_ids.astype(jnp.int32).reshape(-1)
    te = teacher_ids.astype(jnp.int32).reshape(-1)
    sc = school_ids.astype(jnp.int32).reshape(-1)

    bk_p, co_p, st_p, zi_p, te_p, sc_p, n0 = _sc_pool()(
        bk1, bk2, book_table,
        co, country_table, st, state_table, zi, zipcode_table,
        te, teacher_table, sc, school_table)

    th_p, ca_p, sk_p = _counts_pool(
        last_theme_ids.astype(jnp.int32), last_category_ids.astype(jnp.int32),
        last_reading_skills_id.astype(jnp.int32),
        theme_table, category_table, skill_table)

    pooled = (bk_p, th_p, ca_p, sk_p, co_p, st_p, zi_p, te_p, sc_p)
    t0p = jnp.broadcast_to(book_table[0:1], (8, D))
    b1b = jnp.broadcast_to(b1.reshape(1, -1), (8, 256))
    b2b = jnp.broadcast_to(b2.reshape(1, -1), (8, 64))
    return _mlp(pooled, user_features, n0, t0p, W1, b1b, W2, b2b)


# speedup vs baseline: 5.8010x; 5.8010x over previous
import functools

import jax
import jax.numpy as jnp
import numpy as np
from jax import lax
from jax.experimental import pallas as pl
from jax.experimental.pallas import tpu as pltpu
from jax.experimental.pallas import tpu_sc as plsc

B = 4096
D = 128
S = 50
SP = 52
NW = 32
BPW = B // NW
CPW = BPW // 2
CHUNK = 2 * SP
WIDS = BPW * SP


def _mask_at(idx_ref, pos):
    v = idx_ref[pl.ds(pos, 16)]
    return jnp.where(v[0] != 0, jnp.float32(1.0), jnp.float32(0.0))


NBUF = 4
INV50 = 1.0 / 50.0


def _sc_body(bk1, bk2, bk_t,
             co_i, co_t, st_i, st_t, zi_i, zi_t, te_i, te_t, sc_i, sc_t,
             bk_o, co_o, st_o, zi_o, te_o, sc_o, n0_o,
             idx1d, idx2d, rows0, rows1, rows2, rows3, acc, acc2, n0buf,
             sidx0, sidx1, sidx2, sidx3, sidx4,
             sem0, sem1, sem2, sem3):
    wid = lax.axis_index("s") * 2 + lax.axis_index("c")
    base = wid * BPW
    rows = (rows0, rows1, rows2, rows3)
    sems = (sem0, sem1, sem2, sem3)
    lanes = lax.iota(jnp.int32, 16)

    def do_seq(flat_hbm, ids2d_hbm, table_hbm, out_hbm, n0_hbm):
        pltpu.sync_copy(flat_hbm.at[pl.ds(base * SP, WIDS)],
                        idx1d.at[pl.ds(0, WIDS)])
        pltpu.sync_copy(ids2d_hbm.at[pl.ds(wid * CPW, CPW)], idx2d)

        for b in range(NBUF):
            pltpu.make_async_copy(
                table_hbm.at[idx2d.at[b]], rows[b], sems[b]).start()

        def group(g, _):
            for b in range(NBUF):
                c = g * NBUF + b
                rb = rows[b]
                pltpu.make_async_copy(
                    table_hbm.at[idx2d.at[0]], rb, sems[b]).wait()
                for half in range(2):
                    r = c * 2 + half
                    base_i = c * CHUNK + half * SP

                    def jbody(j, accs, _rb=rb, _half=half):
                        gg = _half * SP + j
                        return tuple(
                            accs[k] + _rb[gg, pl.ds(k * 16, 16)]
                            for k in range(8))

                    accs = lax.fori_loop(
                        0, SP, jbody,
                        tuple(jnp.zeros((16,), jnp.float32) for _ in range(8)),
                        unroll=4)
                    n0 = jnp.zeros((16,), jnp.int32)
                    for t in range(3):
                        v = idx1d[pl.ds(base_i + t * 16, 16)]
                        n0 = n0 + plsc.all_reduce_population_count(v == 0)
                    v3 = idx1d[pl.ds(base_i + 48, 16)]
                    n0 = n0 + plsc.all_reduce_population_count(
                        (v3 == 0) & (lanes < S - 48))
                    n0buf[r, pl.ds(0, 16)] = n0.astype(jnp.float32)
                    g49 = half * SP + (S - 1)
                    for k in range(8):
                        acc[r, pl.ds(k * 16, 16)] = (
                            (accs[k] - 2.0 * rb[g49, pl.ds(k * 16, 16)])
                            * jnp.float32(INV50))
                nxt = c + NBUF

                @pl.when(nxt < CPW)
                def _():
                    pltpu.make_async_copy(
                        table_hbm.at[idx2d.at[nxt]], rb, sems[b]).start()
            return 0

        lax.fori_loop(0, CPW // NBUF, group, 0)
        pltpu.sync_copy(acc, out_hbm.at[pl.ds(base, BPW)])
        pltpu.sync_copy(n0buf, n0_hbm.at[pl.ds(base, BPW)])

    def mask_rows(buf, idx_ref):
        def rbody(g, _):
            m = _mask_at(idx_ref, g)
            for k in range(8):
                buf[g, pl.ds(k * 16, 16)] = buf[g, pl.ds(k * 16, 16)] * m
            return 0
        lax.fori_loop(0, BPW, rbody, 0)

    do_seq(bk1, bk2, bk_t, bk_o, n0_o)

    singles = ((co_i, co_t, co_o, sidx0), (st_i, st_t, st_o, sidx1),
               (zi_i, zi_t, zi_o, sidx2), (te_i, te_t, te_o, sidx3),
               (sc_i, sc_t, sc_o, sidx4))
    bufs = (acc, acc2)
    for f, (ids_hbm, _, _, sx) in enumerate(singles):
        pltpu.sync_copy(ids_hbm.at[pl.ds(base, BPW)], sx.at[pl.ds(0, BPW)])
    pltpu.make_async_copy(
        singles[0][1].at[sidx0.at[pl.ds(0, BPW)]], bufs[0], sems[0]).start()
    for f, (_, table_hbm, out_hbm, sx) in enumerate(singles):
        buf = bufs[f % 2]
        pltpu.make_async_copy(
            table_hbm.at[sx.at[pl.ds(0, BPW)]], buf, sems[f % 2]).wait()
        if f + 1 < len(singles):
            nx = singles[f + 1]
            pltpu.make_async_copy(
                nx[1].at[nx[3].at[pl.ds(0, BPW)]],
                bufs[(f + 1) % 2], sems[(f + 1) % 2]).start()
        mask_rows(buf, sx)
        pltpu.sync_copy(buf, out_hbm.at[pl.ds(base, BPW)])


@functools.cache
def _sc_pool():
    return pl.kernel(
        _sc_body,
        out_type=[jax.ShapeDtypeStruct((B, D), jnp.float32) for _ in range(6)]
        + [jax.ShapeDtypeStruct((B, 16), jnp.float32)],
        mesh=plsc.VectorSubcoreMesh(core_axis_name="c", subcore_axis_name="s"),
        scratch_types=(
            [pltpu.VMEM((WIDS + 16,), jnp.int32),
             pltpu.VMEM((CPW, CHUNK), jnp.int32)]
            + [pltpu.VMEM((CHUNK, D) if b < NBUF else (8, D), jnp.float32)
               for b in range(4)]
            + [pltpu.VMEM((BPW, D), jnp.float32),
               pltpu.VMEM((BPW, D), jnp.float32),
               pltpu.VMEM((BPW, 16), jnp.float32)]
            + [pltpu.VMEM((BPW + 16,), jnp.int32) for _ in range(5)]
            + [pltpu.SemaphoreType.DMA for _ in range(4)]
        ),
        compiler_params=pltpu.CompilerParams(needs_layout_passes=False),
    )


V_SMALL = 1000
CBLK = 128


def _counts_pool_body(th_ids, ca_ids, sk_ids, th_t, ca_t, sk_t,
                      th_o, ca_o, sk_o, cnt):
    vcol = lax.broadcasted_iota(jnp.int32, (CBLK, V_SMALL), 1)
    colmask = (vcol > 0).astype(jnp.float32)
    for ids_ref, t_ref, o_ref in ((th_ids, th_t, th_o),
                                  (ca_ids, ca_t, ca_o),
                                  (sk_ids, sk_t, sk_o)):
        cnt[...] = jnp.zeros((CBLK, V_SMALL), jnp.float32)
        for j in range(S):
            cnt[...] = cnt[...] + (
                ids_ref[:, j:j + 1] == vcol).astype(jnp.float32)
        o_ref[...] = jnp.dot(cnt[...] * colmask, t_ref[...],
                             preferred_element_type=jnp.float32) * (1.0 / S)


def _counts_pool(th_ids, ca_ids, sk_ids, th_t, ca_t, sk_t):
    grid = (B // CBLK,)
    return pl.pallas_call(
        _counts_pool_body,
        grid=grid,
        in_specs=[pl.BlockSpec((CBLK, S), lambda i: (i, 0))] * 3
        + [pl.BlockSpec((V_SMALL, D), lambda i: (0, 0))] * 3,
        out_specs=[pl.BlockSpec((CBLK, D), lambda i: (i, 0))] * 3,
        out_shape=[jax.ShapeDtypeStruct((B, D), jnp.float32)] * 3,
        scratch_shapes=[pltpu.VMEM((CBLK, V_SMALL), jnp.float32)],
    )(th_ids, ca_ids, sk_ids, th_t, ca_t, sk_t)


def _mlp_body(p_refs, uf_ref, n0_ref, t0p_ref, w1_ref, b1_ref, w2_ref,
              b2_ref, out_ref):
    h = jnp.dot(uf_ref[...], w1_ref[pl.ds(9 * D, D), :],
                preferred_element_type=jnp.float32)
    for f in range(9):
        h = h + jnp.dot(p_refs[f][...], w1_ref[pl.ds(f * D, D), :],
                        preferred_element_type=jnp.float32)
    t0w = jnp.dot(t0p_ref[0:1, :], w1_ref[pl.ds(0, D), :],
                  preferred_element_type=jnp.float32)
    h = h - (n0_ref[:, 0:1] * jnp.float32(INV50)) * t0w
    h = jnp.maximum(h + b1_ref[0:1, :], 0.0)
    out_ref[...] = (jnp.dot(h, w2_ref[...], preferred_element_type=jnp.float32)
                    + b2_ref[0:1, :])


def _mlp(pooled, uf, n0, t0p, w1, b1, w2, b2):
    blk = 512
    grid = (B // blk,)
    p_specs = [pl.BlockSpec((blk, D), lambda i: (i, 0)) for _ in range(9)]

    def body(*refs):
        _mlp_body(list(refs[0:9]), refs[9], refs[10], refs[11], refs[12],
                  refs[13], refs[14], refs[15], refs[16])

    return pl.pallas_call(
        body,
        grid=grid,
        in_specs=p_specs + [
            pl.BlockSpec((blk, D), lambda i: (i, 0)),
            pl.BlockSpec((blk, 16), lambda i: (i, 0)),
            pl.BlockSpec((8, D), lambda i: (0, 0)),
            pl.BlockSpec((10 * D, 256), lambda i: (0, 0)),
            pl.BlockSpec((8, 256), lambda i: (0, 0)),
            pl.BlockSpec((256, 64), lambda i: (0, 0)),
            pl.BlockSpec((8, 64), lambda i: (0, 0)),
        ],
        out_specs=pl.BlockSpec((blk, 64), lambda i: (i, 0)),
        out_shape=jax.ShapeDtypeStruct((B, 64), jnp.float32),
    )(*pooled, uf, n0, t0p, w1, b1, w2, b2)


def kernel(last_book_ids, last_book_mask, last_theme_ids, last_theme_mask,
           last_category_ids, last_category_mask, last_reading_skills_id,
           last_reading_skills_mask, country_ids, country_mask, state_ids,
           state_mask, zipcode_ids, zipcode_mask, teacher_ids, teacher_mask,
           school_ids, school_mask, user_features, book_table, theme_table,
           category_table, skill_table, country_table, state_table,
           zipcode_table, teacher_table, school_table, W1, b1, W2, b2):
    def seq_prep(ids):
        p = ids.astype(jnp.int32)
        p = jnp.concatenate([p, p[:, S - 1:S], p[:, S - 1:S]], axis=1)
        return p.reshape(-1), p.reshape(-1, CHUNK)

    bk1, bk2 = seq_prep(last_book_ids)
    co = country_ids.astype(jnp.int32).reshape(-1)
    st = state_ids.astype(jnp.int32).reshape(-1)
    zi = zipcode_ids.astype(jnp.int32).reshape(-1)
    te = teacher_ids.astype(jnp.int32).reshape(-1)
    sc = school_ids.astype(jnp.int32).reshape(-1)

    bk_p, co_p, st_p, zi_p, te_p, sc_p, n0 = _sc_pool()(
        bk1, bk2, book_table,
        co, country_table, st, state_table, zi, zipcode_table,
        te, teacher_table, sc, school_table)

    th_p, ca_p, sk_p = _counts_pool(
        last_theme_ids.astype(jnp.int32), last_category_ids.astype(jnp.int32),
        last_reading_skills_id.astype(jnp.int32),
        theme_table, category_table, skill_table)

    pooled = (bk_p, th_p, ca_p, sk_p, co_p, st_p, zi_p, te_p, sc_p)
    t0p = jnp.broadcast_to(book_table[0:1], (8, D))
    b1b = jnp.broadcast_to(b1.reshape(1, -1), (8, 256))
    b2b = jnp.broadcast_to(b2.reshape(1, -1), (8, 64))
    return _mlp(pooled, user_features, n0, t0p, W1, b1b, W2, b2b)


# speedup vs baseline: 5.8028x; 1.0003x over previous
import functools

import jax
import jax.numpy as jnp
import numpy as np
from jax import lax
from jax.experimental import pallas as pl
from jax.experimental.pallas import tpu as pltpu
from jax.experimental.pallas import tpu_sc as plsc

B = 4096
D = 128
S = 50
SP = 52
NW = 32
BPW = B // NW
CPW = BPW // 2
CHUNK = 2 * SP
WIDS = BPW * SP


def _mask_at(idx_ref, pos):
    v = idx_ref[pl.ds(pos, 16)]
    return jnp.where(v[0] != 0, jnp.float32(1.0), jnp.float32(0.0))


NBUF = 4
INV50 = 1.0 / 50.0


def _sc_body(bk1, bk2, bk_t,
             co_i, co_t, st_i, st_t, zi_i, zi_t, te_i, te_t, sc_i, sc_t,
             bk_o, co_o, st_o, zi_o, te_o, sc_o, n0_o,
             idx1d, idx2d, rows0, rows1, rows2, rows3, acc, acc2, n0buf,
             sidx0, sidx1, sidx2, sidx3, sidx4,
             sem0, sem1, sem2, sem3, sem4):
    wid = lax.axis_index("s") * 2 + lax.axis_index("c")
    base = wid * BPW
    rows = (rows0, rows1, rows2, rows3)
    sems = (sem0, sem1, sem2, sem3)
    lanes = lax.iota(jnp.int32, 16)

    def do_seq(flat_hbm, ids2d_hbm, table_hbm, out_hbm, n0_hbm):
        pltpu.sync_copy(flat_hbm.at[pl.ds(base * SP, WIDS)],
                        idx1d.at[pl.ds(0, WIDS)])
        pltpu.sync_copy(ids2d_hbm.at[pl.ds(wid * CPW, CPW)], idx2d)

        for b in range(NBUF):
            pltpu.make_async_copy(
                table_hbm.at[idx2d.at[b]], rows[b], sems[b]).start()

        def group(g, _):
            for b in range(NBUF):
                c = g * NBUF + b
                rb = rows[b]
                pltpu.make_async_copy(
                    table_hbm.at[idx2d.at[0]], rb, sems[b]).wait()
                for half in range(2):
                    r = c * 2 + half
                    base_i = c * CHUNK + half * SP

                    def jbody(j, accs, _rb=rb, _half=half):
                        gg = _half * SP + j
                        return tuple(
                            accs[k] + _rb[gg, pl.ds(k * 16, 16)]
                            for k in range(8))

                    accs = lax.fori_loop(
                        0, SP, jbody,
                        tuple(jnp.zeros((16,), jnp.float32) for _ in range(8)),
                        unroll=4)
                    n0 = jnp.zeros((16,), jnp.int32)
                    for t in range(3):
                        v = idx1d[pl.ds(base_i + t * 16, 16)]
                        n0 = n0 + plsc.all_reduce_population_count(v == 0)
                    v3 = idx1d[pl.ds(base_i + 48, 16)]
                    n0 = n0 + plsc.all_reduce_population_count(
                        (v3 == 0) & (lanes < S - 48))
                    n0buf[r, pl.ds(0, 16)] = n0.astype(jnp.float32)
                    g49 = half * SP + (S - 1)
                    for k in range(8):
                        acc[r, pl.ds(k * 16, 16)] = (
                            (accs[k] - 2.0 * rb[g49, pl.ds(k * 16, 16)])
                            * jnp.float32(INV50))
                nxt = c + NBUF

                @pl.when(nxt < CPW)
                def _():
                    pltpu.make_async_copy(
                        table_hbm.at[idx2d.at[nxt]], rb, sems[b]).start()
            return 0

        lax.fori_loop(0, CPW // NBUF, group, 0)
        pltpu.sync_copy(acc, out_hbm.at[pl.ds(base, BPW)])
        pltpu.sync_copy(n0buf, n0_hbm.at[pl.ds(base, BPW)])

    def mask_rows(buf, idx_ref):
        def rbody(g, _):
            m = _mask_at(idx_ref, g)
            for k in range(8):
                buf[g, pl.ds(k * 16, 16)] = buf[g, pl.ds(k * 16, 16)] * m
            return 0
        lax.fori_loop(0, BPW, rbody, 0)

    singles = ((co_i, co_t, co_o, sidx0), (st_i, st_t, st_o, sidx1),
               (zi_i, zi_t, zi_o, sidx2), (te_i, te_t, te_o, sidx3),
               (sc_i, sc_t, sc_o, sidx4))
    bufs = (acc2, acc)
    for f, (ids_hbm, _, _, sx) in enumerate(singles):
        pltpu.sync_copy(ids_hbm.at[pl.ds(base, BPW)], sx.at[pl.ds(0, BPW)])
    def ssem(f):
        return sem4 if f == 0 else sems[f % 2]

    pltpu.make_async_copy(
        singles[0][1].at[sidx0.at[pl.ds(0, BPW)]], bufs[0], ssem(0)).start()

    do_seq(bk1, bk2, bk_t, bk_o, n0_o)
    for f, (_, table_hbm, out_hbm, sx) in enumerate(singles):
        buf = bufs[f % 2]
        pltpu.make_async_copy(
            table_hbm.at[sx.at[pl.ds(0, BPW)]], buf, ssem(f)).wait()
        if f + 1 < len(singles):
            nx = singles[f + 1]
            pltpu.make_async_copy(
                nx[1].at[nx[3].at[pl.ds(0, BPW)]],
                bufs[(f + 1) % 2], ssem(f + 1)).start()
        mask_rows(buf, sx)
        pltpu.sync_copy(buf, out_hbm.at[pl.ds(base, BPW)])


@functools.cache
def _sc_pool():
    return pl.kernel(
        _sc_body,
        out_type=[jax.ShapeDtypeStruct((B, D), jnp.float32) for _ in range(6)]
        + [jax.ShapeDtypeStruct((B, 16), jnp.float32)],
        mesh=plsc.VectorSubcoreMesh(core_axis_name="c", subcore_axis_name="s"),
        scratch_types=(
            [pltpu.VMEM((WIDS + 16,), jnp.int32),
             pltpu.VMEM((CPW, CHUNK), jnp.int32)]
            + [pltpu.VMEM((CHUNK, D) if b < NBUF else (8, D), jnp.float32)
               for b in range(4)]
            + [pltpu.VMEM((BPW, D), jnp.float32),
               pltpu.VMEM((BPW, D), jnp.float32),
               pltpu.VMEM((BPW, 16), jnp.float32)]
            + [pltpu.VMEM((BPW + 16,), jnp.int32) for _ in range(5)]
            + [pltpu.SemaphoreType.DMA for _ in range(5)]
        ),
        compiler_params=pltpu.CompilerParams(needs_layout_passes=False),
    )


V_SMALL = 1000
CBLK = 128


def _counts_pool_body(th_ids, ca_ids, sk_ids, th_t, ca_t, sk_t,
                      th_o, ca_o, sk_o, cnt):
    vcol = lax.broadcasted_iota(jnp.int32, (CBLK, V_SMALL), 1)
    colmask = (vcol > 0).astype(jnp.float32)
    for ids_ref, t_ref, o_ref in ((th_ids, th_t, th_o),
                                  (ca_ids, ca_t, ca_o),
                                  (sk_ids, sk_t, sk_o)):
        cnt[...] = jnp.zeros((CBLK, V_SMALL), jnp.float32)
        for j in range(S):
            cnt[...] = cnt[...] + (
                ids_ref[:, j:j + 1] == vcol).astype(jnp.float32)
        o_ref[...] = jnp.dot(cnt[...] * colmask, t_ref[...],
                             preferred_element_type=jnp.float32) * (1.0 / S)


def _counts_pool(th_ids, ca_ids, sk_ids, th_t, ca_t, sk_t):
    grid = (B // CBLK,)
    return pl.pallas_call(
        _counts_pool_body,
        grid=grid,
        in_specs=[pl.BlockSpec((CBLK, S), lambda i: (i, 0))] * 3
        + [pl.BlockSpec((V_SMALL, D), lambda i: (0, 0))] * 3,
        out_specs=[pl.BlockSpec((CBLK, D), lambda i: (i, 0))] * 3,
        out_shape=[jax.ShapeDtypeStruct((B, D), jnp.float32)] * 3,
        scratch_shapes=[pltpu.VMEM((CBLK, V_SMALL), jnp.float32)],
    )(th_ids, ca_ids, sk_ids, th_t, ca_t, sk_t)


def _mlp_body(p_refs, uf_ref, n0_ref, t0p_ref, w1_ref, b1_ref, w2_ref,
              b2_ref, out_ref):
    h = jnp.dot(uf_ref[...], w1_ref[pl.ds(9 * D, D), :],
                preferred_element_type=jnp.float32)
    for f in range(9):
        h = h + jnp.dot(p_refs[f][...], w1_ref[pl.ds(f * D, D), :],
                        preferred_element_type=jnp.float32)
    t0w = jnp.dot(t0p_ref[0:1, :], w1_ref[pl.ds(0, D), :],
                  preferred_element_type=jnp.float32)
    h = h - (n0_ref[:, 0:1] * jnp.float32(INV50)) * t0w
    h = jnp.maximum(h + b1_ref[0:1, :], 0.0)
    out_ref[...] = (jnp.dot(h, w2_ref[...], preferred_element_type=jnp.float32)
                    + b2_ref[0:1, :])


def _mlp(pooled, uf, n0, t0p, w1, b1, w2, b2):
    blk = 512
    grid = (B // blk,)
    p_specs = [pl.BlockSpec((blk, D), lambda i: (i, 0)) for _ in range(9)]

    def body(*refs):
        _mlp_body(list(refs[0:9]), refs[9], refs[10], refs[11], refs[12],
                  refs[13], refs[14], refs[15], refs[16])

    return pl.pallas_call(
        body,
        grid=grid,
        in_specs=p_specs + [
            pl.BlockSpec((blk, D), lambda i: (i, 0)),
            pl.BlockSpec((blk, 16), lambda i: (i, 0)),
            pl.BlockSpec((8, D), lambda i: (0, 0)),
            pl.BlockSpec((10 * D, 256), lambda i: (0, 0)),
            pl.BlockSpec((8, 256), lambda i: (0, 0)),
            pl.BlockSpec((256, 64), lambda i: (0, 0)),
            pl.BlockSpec((8, 64), lambda i: (0, 0)),
        ],
        out_specs=pl.BlockSpec((blk, 64), lambda i: (i, 0)),
        out_shape=jax.ShapeDtypeStruct((B, 64), jnp.float32),
    )(*pooled, uf, n0, t0p, w1, b1, w2, b2)


def kernel(last_book_ids, last_book_mask, last_theme_ids, last_theme_mask,
           last_category_ids, last_category_mask, last_reading_skills_id,
           last_reading_skills_mask, country_ids, country_mask, state_ids,
           state_mask, zipcode_ids, zipcode_mask, teacher_ids, teacher_mask,
           school_ids, school_mask, user_features, book_table, theme_table,
           category_table, skill_table, country_table, state_table,
           zipcode_table, teacher_table, school_table, W1, b1, W2, b2):
    def seq_prep(ids):
        p = ids.astype(jnp.int32)
        p = jnp.concatenate([p, p[:, S - 1:S], p[:, S - 1:S]], axis=1)
        return p.reshape(-1), p.reshape(-1, CHUNK)

    bk1, bk2 = seq_prep(last_book_ids)
    co = country_ids.astype(jnp.int32).reshape(-1)
    st = state_ids.astype(jnp.int32).reshape(-1)
    zi = zipcode_ids.astype(jnp.int32).reshape(-1)
    te = teacher_ids.astype(jnp.int32).reshape(-1)
    sc = school_ids.astype(jnp.int32).reshape(-1)

    bk_p, co_p, st_p, zi_p, te_p, sc_p, n0 = _sc_pool()(
        bk1, bk2, book_table,
        co, country_table, st, state_table, zi, zipcode_table,
        te, teacher_table, sc, school_table)

    th_p, ca_p, sk_p = _counts_pool(
        last_theme_ids.astype(jnp.int32), last_category_ids.astype(jnp.int32),
        last_reading_skills_id.astype(jnp.int32),
        theme_table, category_table, skill_table)

    pooled = (bk_p, th_p, ca_p, sk_p, co_p, st_p, zi_p, te_p, sc_p)
    t0p = jnp.broadcast_to(book_table[0:1], (8, D))
    b1b = jnp.broadcast_to(b1.reshape(1, -1), (8, 256))
    b2b = jnp.broadcast_to(b2.reshape(1, -1), (8, 64))
    return _mlp(pooled, user_features, n0, t0p, W1, b1b, W2, b2b)
